# Initial kernel scaffold; baseline (speedup 1.0000x reference)
import jax
import jax.numpy as jnp
from jax.experimental import pallas as pl


def kernel(x, edge_index, W1, b1, W2, b2):
    raise NotImplementedError("write your pallas kernel here")


# speedup vs baseline: 33.4331x; 33.4331x over previous
import functools

import jax
import jax.numpy as jnp
from jax import lax
from jax.experimental import pallas as pl
from jax.experimental.pallas import tpu as pltpu
from jax.experimental.pallas import tpu_sc as plsc

N = 50000
E = 800000
P = 51200
D = 16
NC, NS = 2, 16
NW = NC * NS
EPW = E // NW
C = 1000
SL = P // NS
R = 1024


def _mesh():
    return plsc.VectorSubcoreMesh(
        core_axis_name="c", subcore_axis_name="s", num_cores=NC, num_subcores=NS
    )


def _deg_body(dst_hbm, ones_hbm, zeros_hbm, out_hbm, idx_v, ones_v, accum_sh):
    c = lax.axis_index("c")
    s = lax.axis_index("s")
    pltpu.sync_copy(zeros_hbm, accum_sh.at[pl.ds(s * SL, SL)])
    pltpu.sync_copy(ones_hbm, ones_v)
    plsc.subcore_barrier()
    base = (c * NS + s) * EPW

    def step(j, carry):
        pltpu.sync_copy(dst_hbm.at[pl.ds(base + j * C, C)], idx_v)
        pltpu.sync_copy(ones_v, accum_sh.at[idx_v], add=True)
        return carry

    lax.fori_loop(0, EPW // C, step, 0)
    plsc.subcore_barrier()
    pltpu.sync_copy(accum_sh.at[pl.ds(s * SL, SL)],
                    out_hbm.at[pl.ds(c * P + s * SL, SL)])


def _agg_body(table_hbm, src_hbm, dst_hbm, zeros_hbm, out_hbm,
              idx_s, idx_d, rows_v, accum_sh):
    c = lax.axis_index("c")
    s = lax.axis_index("s")
    pltpu.sync_copy(zeros_hbm, accum_sh.at[pl.ds(s * SL, SL)])
    plsc.subcore_barrier()
    base = (c * NS + s) * EPW

    def step(j, carry):
        pltpu.sync_copy(src_hbm.at[pl.ds(base + j * C, C)], idx_s)
        pltpu.sync_copy(dst_hbm.at[pl.ds(base + j * C, C)], idx_d)
        pltpu.sync_copy(table_hbm.at[idx_s], rows_v)
        pltpu.sync_copy(rows_v, accum_sh.at[idx_d], add=True)
        return carry

    lax.fori_loop(0, EPW // C, step, 0)
    plsc.subcore_barrier()
    pltpu.sync_copy(accum_sh.at[pl.ds(s * SL, SL)],
                    out_hbm.at[pl.ds(c * P + s * SL, SL)])


_SC_PARAMS = pltpu.CompilerParams(use_tc_tiling_on_sc=False)


def _deg(dst):
    k = functools.partial(
        pl.kernel,
        out_type=jax.ShapeDtypeStruct((NC * P,), jnp.float32),
        mesh=_mesh(),
        compiler_params=_SC_PARAMS,
        scratch_types=[
            pltpu.VMEM((C,), jnp.int32),
            pltpu.VMEM((C,), jnp.float32),
            pltpu.VMEM_SHARED((P,), jnp.float32),
        ],
    )(_deg_body)
    return k(dst, jnp.ones((C,), jnp.float32), jnp.zeros((SL,), jnp.float32))


def _agg(table, src, dst):
    k = functools.partial(
        pl.kernel,
        out_type=jax.ShapeDtypeStruct((NC * P, D), jnp.float32),
        mesh=_mesh(),
        compiler_params=_SC_PARAMS,
        scratch_types=[
            pltpu.VMEM((C,), jnp.int32),
            pltpu.VMEM((C,), jnp.int32),
            pltpu.VMEM((C, D), jnp.float32),
            pltpu.VMEM_SHARED((P, D), jnp.float32),
        ],
    )(_agg_body)
    return k(table, src, dst, jnp.zeros((SL, D), jnp.float32))


def _tc1_body(x_ref, w_ref, dc_ref, hs_ref, dinv_ref):
    deg = dc_ref[:, 0:1] + dc_ref[:, 1:2] + 1.0
    dinv = lax.rsqrt(deg)
    h = jnp.dot(x_ref[...], w_ref[...], preferred_element_type=jnp.float32)
    hs_ref[...] = dinv * h
    dinv_ref[...] = dinv


def _tc2_body(p0_ref, p1_ref, hs_ref, dinv_ref, b_ref, out_ref):
    dinv = dinv_ref[...]
    pre = dinv * (p0_ref[...] + p1_ref[...] + hs_ref[...]) + b_ref[...]
    out_ref[...] = dinv * jnp.maximum(pre, 0.0)


def _tc3_body(q0_ref, q1_ref, hs_ref, dinv_ref, w_ref, b_ref, out_ref):
    aggv = dinv_ref[...] * (q0_ref[...] + q1_ref[...] + hs_ref[...])
    o = jnp.dot(aggv, w_ref[...], preferred_element_type=jnp.float32)
    o = o + b_ref[...]
    m = jnp.max(o, axis=1, keepdims=True)
    lse = m + jnp.log(jnp.sum(jnp.exp(o - m), axis=1, keepdims=True))
    out_ref[...] = o - lse


def _rows(block_cols):
    return pl.BlockSpec((R, block_cols), lambda i: (i, 0))


def _full(shape):
    return pl.BlockSpec(shape, lambda i: tuple(0 for _ in shape))


def _tc1(x_pad, W1, deg_cols):
    d_in = x_pad.shape[1]
    return pl.pallas_call(
        _tc1_body,
        grid=(P // R,),
        in_specs=[_rows(d_in), _full(W1.shape), _rows(2)],
        out_specs=[_rows(D), _rows(1)],
        out_shape=[
            jax.ShapeDtypeStruct((P, D), jnp.float32),
            jax.ShapeDtypeStruct((P, 1), jnp.float32),
        ],
    )(x_pad, W1, deg_cols)


def _tc2(p0, p1, hs1, dinv, b1):
    return pl.pallas_call(
        _tc2_body,
        grid=(P // R,),
        in_specs=[_rows(D), _rows(D), _rows(D), _rows(1), _full((1, D))],
        out_specs=_rows(D),
        out_shape=jax.ShapeDtypeStruct((P, D), jnp.float32),
    )(p0, p1, hs1, dinv, b1)


def _tc3(q0, q1, hs2, dinv, W2, b2):
    d_out = W2.shape[1]
    return pl.pallas_call(
        _tc3_body,
        grid=(P // R,),
        in_specs=[_rows(D), _rows(D), _rows(D), _rows(1), _full(W2.shape),
                  _full((1, d_out))],
        out_specs=_rows(d_out),
        out_shape=jax.ShapeDtypeStruct((P, d_out), jnp.float32),
    )(q0, q1, hs2, dinv, W2, b2)


def kernel(x, edge_index, W1, b1, W2, b2):
    ei = edge_index.astype(jnp.int32)
    src, dst = ei[0], ei[1]
    x_pad = jnp.zeros((P, x.shape[1]), x.dtype).at[:N].set(x)

    degp = _deg(dst)
    deg_cols = jnp.stack([degp[:P], degp[P:]], axis=1)

    hs1, dinv = _tc1(x_pad, W1, deg_cols)
    a1 = _agg(hs1, src, dst)
    hs2 = _tc2(a1[:P], a1[P:], hs1, dinv, b1.reshape(1, D))
    a2 = _agg(hs2, src, dst)
    out = _tc3(a2[:P], a2[P:], hs2, dinv, W2, b2.reshape(1, W2.shape[1]))
    return out[:N]


# speedup vs baseline: 53.3980x; 1.5972x over previous
import functools

import jax
import jax.numpy as jnp
from jax import lax
from jax.experimental import pallas as pl
from jax.experimental.pallas import tpu as pltpu
from jax.experimental.pallas import tpu_sc as plsc

N = 50000
E = 800000
P = 51200
D = 16
NC, NS = 2, 16
NW = NC * NS
EPW = E // NW
C = 1000
G = EPW // C
SL = P // NS
R = 2000


def _mesh():
    return plsc.VectorSubcoreMesh(
        core_axis_name="c", subcore_axis_name="s", num_cores=NC, num_subcores=NS
    )


_SC_PARAMS = pltpu.CompilerParams(use_tc_tiling_on_sc=False)


def _deg_body(dst_hbm, ones_hbm, zeros_hbm, out_hbm, idx_v, ones_v, accum_sh):
    c = lax.axis_index("c")
    s = lax.axis_index("s")
    pltpu.sync_copy(zeros_hbm, accum_sh.at[pl.ds(s * SL, SL)])
    pltpu.sync_copy(ones_hbm, ones_v)
    plsc.subcore_barrier()
    base = (c * NS + s) * EPW

    def step(j, carry):
        pltpu.sync_copy(dst_hbm.at[pl.ds(base + j * C, C)], idx_v)
        pltpu.sync_copy(ones_v, accum_sh.at[idx_v], add=True)
        return carry

    lax.fori_loop(0, G, step, 0)
    plsc.subcore_barrier()
    pltpu.sync_copy(accum_sh.at[pl.ds(s * SL, SL)],
                    out_hbm.at[pl.ds(c * P + s * SL, SL)])


def _agg_body(table_hbm, src_hbm, dst_hbm, zeros_hbm, out_hbm,
              sbuf0, sbuf1, dbuf0, dbuf1, rows0, rows1,
              ssrc0, ssrc1, sd0, sd1, sg0, sg1, ss0, ss1, accum_sh):
    c = lax.axis_index("c")
    s = lax.axis_index("s")
    sbuf = [sbuf0, sbuf1]
    dbuf = [dbuf0, dbuf1]
    rows = [rows0, rows1]
    ssrc = [ssrc0, ssrc1]
    sd = [sd0, sd1]
    sg = [sg0, sg1]
    ss = [ss0, ss1]

    pltpu.sync_copy(zeros_hbm, accum_sh.at[pl.ds(s * SL, SL)])
    plsc.subcore_barrier()
    base = (c * NS + s) * EPW

    def _wait_scatter(b):
        pltpu.make_async_copy(rows[b], accum_sh.at[dbuf[b]], ss[b]).wait()

    def _wait_src(b, j):
        pltpu.make_async_copy(
            src_hbm.at[pl.ds(base + j * C, C)], sbuf[b], ssrc[b]).wait()

    for b in (0, 1):
        pltpu.async_copy(src_hbm.at[pl.ds(base + b * C, C)], sbuf[b], ssrc[b])

    def pair(g, carry):
        for b in (0, 1):
            j = 2 * g + b

            @pl.when(g >= 1)
            def _():
                _wait_scatter(b)

            _wait_src(b, j)
            gat = pltpu.async_copy(table_hbm.at[sbuf[b]], rows[b], sg[b])
            dld = pltpu.async_copy(
                dst_hbm.at[pl.ds(base + j * C, C)], dbuf[b], sd[b])
            gat.wait()
            dld.wait()
            pltpu.async_copy(rows[b], accum_sh.at[dbuf[b]], ss[b], add=True)
            if b == 0:
                pltpu.async_copy(
                    src_hbm.at[pl.ds(base + (j + 2) * C, C)], sbuf[b], ssrc[b])
            else:

                @pl.when(g < G // 2 - 1)
                def _():
                    pltpu.async_copy(
                        src_hbm.at[pl.ds(base + (j + 2) * C, C)],
                        sbuf[b], ssrc[b])
        return carry

    lax.fori_loop(0, G // 2, pair, 0)

    _wait_scatter(0)
    _wait_src(0, G - 1)
    gat = pltpu.async_copy(table_hbm.at[sbuf[0]], rows[0], sg[0])
    dld = pltpu.async_copy(
        dst_hbm.at[pl.ds(base + (G - 1) * C, C)], dbuf[0], sd[0])
    gat.wait()
    dld.wait()
    pltpu.async_copy(rows[0], accum_sh.at[dbuf[0]], ss[0], add=True)
    _wait_scatter(0)
    _wait_scatter(1)

    plsc.subcore_barrier()
    pltpu.sync_copy(accum_sh.at[pl.ds(s * SL, SL)],
                    out_hbm.at[pl.ds(c * P + s * SL, SL)])


def _deg(dst):
    k = functools.partial(
        pl.kernel,
        out_type=jax.ShapeDtypeStruct((NC * P,), jnp.float32),
        mesh=_mesh(),
        compiler_params=_SC_PARAMS,
        scratch_types=[
            pltpu.VMEM((C,), jnp.int32),
            pltpu.VMEM((C,), jnp.float32),
            pltpu.VMEM_SHARED((P,), jnp.float32),
        ],
    )(_deg_body)
    return k(dst, jnp.ones((C,), jnp.float32), jnp.zeros((SL,), jnp.float32))


def _agg(table, src, dst):
    k = functools.partial(
        pl.kernel,
        out_type=jax.ShapeDtypeStruct((NC * P, D), jnp.float32),
        mesh=_mesh(),
        compiler_params=_SC_PARAMS,
        scratch_types=[
            pltpu.VMEM((C,), jnp.int32),
            pltpu.VMEM((C,), jnp.int32),
            pltpu.VMEM((C,), jnp.int32),
            pltpu.VMEM((C,), jnp.int32),
            pltpu.VMEM((C, D), jnp.float32),
            pltpu.VMEM((C, D), jnp.float32),
            pltpu.SemaphoreType.DMA,
            pltpu.SemaphoreType.DMA,
            pltpu.SemaphoreType.DMA,
            pltpu.SemaphoreType.DMA,
            pltpu.SemaphoreType.DMA,
            pltpu.SemaphoreType.DMA,
            pltpu.SemaphoreType.DMA,
            pltpu.SemaphoreType.DMA,
            pltpu.VMEM_SHARED((P, D), jnp.float32),
        ],
    )(_agg_body)
    return k(table, src, dst, jnp.zeros((SL, D), jnp.float32))


def _tc1_body(x_ref, w_ref, dc_ref, hs_ref, dinv_ref):
    deg = dc_ref[:, 0:1] + dc_ref[:, 1:2] + 1.0
    dinv = lax.rsqrt(deg)
    h = jnp.dot(x_ref[...], w_ref[...], preferred_element_type=jnp.float32)
    hs_ref[...] = dinv * h
    dinv_ref[...] = dinv


def _tc2_body(p0_ref, p1_ref, hs_ref, dinv_ref, b_ref, out_ref):
    dinv = dinv_ref[...]
    pre = dinv * (p0_ref[0] + p1_ref[0] + hs_ref[...]) + b_ref[...]
    out_ref[...] = dinv * jnp.maximum(pre, 0.0)


def _tc3_body(q0_ref, q1_ref, hs_ref, dinv_ref, w_ref, b_ref, out_ref):
    aggv = dinv_ref[...] * (q0_ref[0] + q1_ref[0] + hs_ref[...])
    o = jnp.dot(aggv, w_ref[...], preferred_element_type=jnp.float32)
    o = o + b_ref[...]
    m = jnp.max(o, axis=1, keepdims=True)
    lse = m + jnp.log(jnp.sum(jnp.exp(o - m), axis=1, keepdims=True))
    out_ref[...] = o - lse


def _rows(block_cols):
    return pl.BlockSpec((R, block_cols), lambda i: (i, 0))


def _plane(which):
    return pl.BlockSpec((1, R, D), lambda i, w=which: (w, i, 0))


def _full(shape):
    return pl.BlockSpec(shape, lambda i: tuple(0 for _ in shape))


def _tc1(x, W1, deg_cols):
    d_in = x.shape[1]
    return pl.pallas_call(
        _tc1_body,
        grid=(N // R,),
        in_specs=[_rows(d_in), _full(W1.shape), _rows(2)],
        out_specs=[_rows(D), _rows(1)],
        out_shape=[
            jax.ShapeDtypeStruct((N, D), jnp.float32),
            jax.ShapeDtypeStruct((N, 1), jnp.float32),
        ],
    )(x, W1, deg_cols)


def _tc2(a1, hs1, dinv, b1):
    return pl.pallas_call(
        _tc2_body,
        grid=(N // R,),
        in_specs=[_plane(0), _plane(1), _rows(D), _rows(1), _full((1, D))],
        out_specs=_rows(D),
        out_shape=jax.ShapeDtypeStruct((N, D), jnp.float32),
    )(a1, a1, hs1, dinv, b1)


def _tc3(a2, hs2, dinv, W2, b2):
    d_out = W2.shape[1]
    return pl.pallas_call(
        _tc3_body,
        grid=(N // R,),
        in_specs=[_plane(0), _plane(1), _rows(D), _rows(1), _full(W2.shape),
                  _full((1, d_out))],
        out_specs=_rows(d_out),
        out_shape=jax.ShapeDtypeStruct((N, d_out), jnp.float32),
    )(a2, a2, hs2, dinv, W2, b2)


def kernel(x, edge_index, W1, b1, W2, b2):
    ei = edge_index.astype(jnp.int32)
    src, dst = ei[0], ei[1]

    degp = _deg(dst)
    deg_cols = jnp.stack([degp[:N], degp[P:P + N]], axis=1)

    hs1, dinv = _tc1(x, W1, deg_cols)
    a1 = _agg(hs1, src, dst).reshape(NC, P, D)
    hs2 = _tc2(a1, hs1, dinv, b1.reshape(1, D))
    a2 = _agg(hs2, src, dst).reshape(NC, P, D)
    out = _tc3(a2, hs2, dinv, W2, b2.reshape(1, W2.shape[1]))
    return out
